# baseline (device time: 471400 ns/iter reference)
import jax
import jax.numpy as jnp
from jax import lax
from jax.experimental import pallas as pl
from jax.experimental.pallas import tpu as pltpu

B = 16
H = 16
D = 64
SCALE = D ** -0.5
CW = D + 2
NC = 1


def kernel(Q, K, V):
    kv_local = K.shape[1]
    rows = kv_local * H // 2 // NC
    Kr = K.reshape(B, kv_local * H // 2, 2 * D)
    Vr = V.reshape(B, kv_local * H // 2, 2 * D)

    def body(q_ref, k_ref, v_ref, out_ref, send_ref, recv_ref,
             send_sem, recv_sem):
        b = pl.program_id(0)
        c = pl.program_id(1)
        nb = pl.num_programs(0)
        nc = pl.num_programs(1)

        qall = q_ref[b, 0, :, :]
        k2d = k_ref[0]
        v2d = v_ref[0]

        par = lax.broadcasted_iota(jnp.int32, (H, 1), 0) & 1
        zero = jnp.zeros_like(qall)
        q2 = jnp.concatenate(
            [jnp.where(par == 0, qall, zero),
             jnp.where(par == 1, qall, zero)],
            axis=1,
        )
        s = lax.dot_general(
            q2, k2d, (((1,), (1,)), ((), ())),
            preferred_element_type=jnp.float32,
        ) * SCALE
        lane = lax.broadcasted_iota(jnp.int32, (H, rows), 1)
        head = lax.broadcasted_iota(jnp.int32, (H, rows), 0)
        s = jnp.where((lane & (H // 2 - 1)) == (head >> 1), s, -1e30)
        m = jnp.max(s, axis=1, keepdims=True)
        p = jnp.exp(s - m)
        l = jnp.sum(p, axis=1, keepdims=True)
        o2 = lax.dot_general(
            p, v2d, (((1,), (0,)), ((), ())),
            preferred_element_type=jnp.float32,
        )
        o = jnp.where(par == 0, o2[:, 0:D], o2[:, D:2 * D])
        row = jnp.concatenate([o, m, l], axis=1)

        @pl.when(c == 0)
        def _():
            send_ref[b, :, pl.ds(0, CW)] = row

        @pl.when(c > 0)
        def _():
            prev = send_ref[b, :, pl.ds(0, CW)]
            o0 = prev[:, 0:D]
            m0 = prev[:, D:D + 1]
            l0 = prev[:, D + 1:D + 2]
            m01 = jnp.maximum(m0, m)
            a0 = jnp.exp(m0 - m01)
            a1 = jnp.exp(m - m01)
            send_ref[b, :, pl.ds(0, CW)] = jnp.concatenate(
                [o0 * a0 + o * a1, m01, l0 * a0 + l * a1], axis=1)

        @pl.when((b == nb - 1) & (c == nc - 1))
        def _():
            my_x = lax.axis_index("x")
            my_y = lax.axis_index("y")
            my_z = lax.axis_index("z")
            nbr = (1 - my_x, my_y, my_z)

            barrier = pltpu.get_barrier_semaphore()
            pl.semaphore_signal(
                barrier, inc=1, device_id=nbr,
                device_id_type=pl.DeviceIdType.MESH,
            )
            pl.semaphore_wait(barrier, 1)

            rdma = pltpu.make_async_remote_copy(
                src_ref=send_ref,
                dst_ref=recv_ref,
                send_sem=send_sem,
                recv_sem=recv_sem,
                device_id=nbr,
                device_id_type=pl.DeviceIdType.MESH,
            )
            rdma.start()
            rdma.wait()

            o_a = send_ref[:, :, 0:D]
            m_a = send_ref[:, :, D:D + 1]
            l_a = send_ref[:, :, D + 1:D + 2]
            o_b = recv_ref[:, :, 0:D]
            m_b = recv_ref[:, :, D:D + 1]
            l_b = recv_ref[:, :, D + 1:D + 2]
            m_n = jnp.maximum(m_a, m_b)
            alpha = jnp.exp(m_a - m_n)
            beta = jnp.exp(m_b - m_n)
            l_n = l_a * alpha + l_b * beta
            out_ref[:, 0, :, :] = (o_a * alpha + o_b * beta) / l_n

    return pl.pallas_call(
        body,
        grid=(B, NC),
        in_specs=[
            pl.BlockSpec((B, 1, H, D), lambda b, c: (0, 0, 0, 0)),
            pl.BlockSpec((1, rows, 2 * D), lambda b, c: (b, c, 0)),
            pl.BlockSpec((1, rows, 2 * D), lambda b, c: (b, c, 0)),
        ],
        out_specs=pl.BlockSpec((B, 1, H, D), lambda b, c: (0, 0, 0, 0)),
        out_shape=jax.ShapeDtypeStruct((B, 1, H, D), jnp.float32),
        scratch_shapes=[
            pltpu.VMEM((B, H, CW), jnp.float32),
            pltpu.VMEM((B, H, CW), jnp.float32),
            pltpu.SemaphoreType.DMA,
            pltpu.SemaphoreType.DMA,
        ],
        compiler_params=pltpu.CompilerParams(collective_id=0),
    )(Q, Kr, Vr)


# device time: 48197 ns/iter; 9.7807x vs baseline; 9.7807x over previous
import jax
import jax.numpy as jnp
from jax import lax
from jax.experimental import pallas as pl
from jax.experimental.pallas import tpu as pltpu

B = 16
H = 16
D = 64
LOG2D = 6
SCALE = D ** -0.5
CW = D + 2


def kernel(Q, K, V):
    kv = K.shape[1]
    Kt = jnp.transpose(K, (0, 2, 3, 1))
    Vt = jnp.transpose(V, (0, 2, 3, 1))

    def body(q_ref, k_ref, v_ref, out_ref, send_ref, recv_ref,
             send_sem, recv_sem):
        b = pl.program_id(0)
        nb = pl.num_programs(0)

        k2d = k_ref[0].reshape(H * D, kv)
        v2d = v_ref[0].reshape(H * D, kv)
        qall = q_ref[b, 0]

        qtile = jnp.concatenate([qall] * H, axis=1)
        col = lax.broadcasted_iota(jnp.int32, (H, H * D), 1)
        rowi = lax.broadcasted_iota(jnp.int32, (H, H * D), 0)
        bd = (col >> LOG2D) == rowi
        qbd = jnp.where(bd, qtile, 0.0)
        s = lax.dot_general(
            qbd, k2d, (((1,), (0,)), ((), ())),
            preferred_element_type=jnp.float32,
        ) * SCALE
        m = jnp.max(s, axis=1, keepdims=True)
        p = jnp.exp(s - m)
        l = jnp.sum(p, axis=1, keepdims=True)
        o_full = lax.dot_general(
            p, v2d, (((1,), (1,)), ((), ())),
            preferred_element_type=jnp.float32,
        )
        o_m = jnp.where(bd, o_full, 0.0)
        o = o_m[:, 0:D]
        for h in range(1, H):
            o = o + o_m[:, h * D:(h + 1) * D]
        row = jnp.concatenate([o, m, l], axis=1)
        send_ref[b, :, pl.ds(0, CW)] = row

        @pl.when(b == nb - 1)
        def _():
            my_x = lax.axis_index("x")
            my_y = lax.axis_index("y")
            my_z = lax.axis_index("z")
            nbr = (1 - my_x, my_y, my_z)

            barrier = pltpu.get_barrier_semaphore()
            pl.semaphore_signal(
                barrier, inc=1, device_id=nbr,
                device_id_type=pl.DeviceIdType.MESH,
            )
            pl.semaphore_wait(barrier, 1)

            rdma = pltpu.make_async_remote_copy(
                src_ref=send_ref,
                dst_ref=recv_ref,
                send_sem=send_sem,
                recv_sem=recv_sem,
                device_id=nbr,
                device_id_type=pl.DeviceIdType.MESH,
            )
            rdma.start()
            rdma.wait()

            o_a = send_ref[:, :, 0:D]
            m_a = send_ref[:, :, D:D + 1]
            l_a = send_ref[:, :, D + 1:D + 2]
            o_b = recv_ref[:, :, 0:D]
            m_b = recv_ref[:, :, D:D + 1]
            l_b = recv_ref[:, :, D + 1:D + 2]
            m_n = jnp.maximum(m_a, m_b)
            alpha = jnp.exp(m_a - m_n)
            beta = jnp.exp(m_b - m_n)
            l_n = l_a * alpha + l_b * beta
            out_ref[:, 0, :, :] = (o_a * alpha + o_b * beta) / l_n

    return pl.pallas_call(
        body,
        grid=(B,),
        in_specs=[
            pl.BlockSpec((B, 1, H, D), lambda b: (0, 0, 0, 0)),
            pl.BlockSpec((1, H, D, kv), lambda b: (b, 0, 0, 0)),
            pl.BlockSpec((1, H, D, kv), lambda b: (b, 0, 0, 0)),
        ],
        out_specs=pl.BlockSpec((B, 1, H, D), lambda b: (0, 0, 0, 0)),
        out_shape=jax.ShapeDtypeStruct((B, 1, H, D), jnp.float32),
        scratch_shapes=[
            pltpu.VMEM((B, H, CW), jnp.float32),
            pltpu.VMEM((B, H, CW), jnp.float32),
            pltpu.SemaphoreType.DMA,
            pltpu.SemaphoreType.DMA,
        ],
        compiler_params=pltpu.CompilerParams(collective_id=0),
    )(Q, Kt, Vt)


# device time: 47382 ns/iter; 9.9489x vs baseline; 1.0172x over previous
import jax
import jax.numpy as jnp
from jax import lax
from jax.experimental import pallas as pl
from jax.experimental.pallas import tpu as pltpu

B = 16
H = 16
D = 64
LOG2D = 6
SCALE = D ** -0.5
CW = D + 2


def kernel(Q, K, V):
    kv = K.shape[1]
    Kt = jnp.transpose(K, (0, 2, 3, 1))
    Vt = jnp.transpose(V, (0, 2, 3, 1))

    def body(q_ref, k_ref, v_ref, out_ref, send_ref, recv_ref,
             send_sems, recv_sems):
        b = pl.program_id(0)
        nb = pl.num_programs(0)

        my_x = lax.axis_index("x")
        my_y = lax.axis_index("y")
        my_z = lax.axis_index("z")
        nbr = (1 - my_x, my_y, my_z)

        def batch_rdma(idx):
            return pltpu.make_async_remote_copy(
                src_ref=send_ref.at[idx],
                dst_ref=recv_ref.at[idx],
                send_sem=send_sems.at[idx],
                recv_sem=recv_sems.at[idx],
                device_id=nbr,
                device_id_type=pl.DeviceIdType.MESH,
            )

        @pl.when(b == 0)
        def _():
            barrier = pltpu.get_barrier_semaphore()
            pl.semaphore_signal(
                barrier, inc=1, device_id=nbr,
                device_id_type=pl.DeviceIdType.MESH,
            )
            pl.semaphore_wait(barrier, 1)

        k2d = k_ref[0].reshape(H * D, kv)
        v2d = v_ref[0].reshape(H * D, kv)
        qall = q_ref[b, 0]

        qtile = jnp.concatenate([qall] * H, axis=1)
        col = lax.broadcasted_iota(jnp.int32, (H, H * D), 1)
        rowi = lax.broadcasted_iota(jnp.int32, (H, H * D), 0)
        bd = (col >> LOG2D) == rowi
        qbd = jnp.where(bd, qtile, 0.0)
        s = lax.dot_general(
            qbd, k2d, (((1,), (0,)), ((), ())),
            preferred_element_type=jnp.float32,
        ) * SCALE
        m = jnp.max(s, axis=1, keepdims=True)
        p = jnp.exp(s - m)
        l = jnp.sum(p, axis=1, keepdims=True)
        o_full = lax.dot_general(
            p, v2d, (((1,), (1,)), ((), ())),
            preferred_element_type=jnp.float32,
        )
        o_m = jnp.where(bd, o_full, 0.0)
        o = o_m[:, 0:D]
        for h in range(1, H):
            o = o + o_m[:, h * D:(h + 1) * D]
        row = jnp.concatenate([o, m, l], axis=1)
        send_ref[b, :, pl.ds(0, CW)] = row

        batch_rdma(b).start()

        @pl.when(b == nb - 1)
        def _():
            for bb in range(B):
                r = batch_rdma(bb)
                r.wait_send()
                r.wait_recv()

            o_a = send_ref[:, :, 0:D]
            m_a = send_ref[:, :, D:D + 1]
            l_a = send_ref[:, :, D + 1:D + 2]
            o_b = recv_ref[:, :, 0:D]
            m_b = recv_ref[:, :, D:D + 1]
            l_b = recv_ref[:, :, D + 1:D + 2]
            m_n = jnp.maximum(m_a, m_b)
            alpha = jnp.exp(m_a - m_n)
            beta = jnp.exp(m_b - m_n)
            l_n = l_a * alpha + l_b * beta
            out_ref[:, 0, :, :] = (o_a * alpha + o_b * beta) / l_n

    return pl.pallas_call(
        body,
        grid=(B,),
        in_specs=[
            pl.BlockSpec((B, 1, H, D), lambda b: (0, 0, 0, 0)),
            pl.BlockSpec((1, H, D, kv), lambda b: (b, 0, 0, 0)),
            pl.BlockSpec((1, H, D, kv), lambda b: (b, 0, 0, 0)),
        ],
        out_specs=pl.BlockSpec((B, 1, H, D), lambda b: (0, 0, 0, 0)),
        out_shape=jax.ShapeDtypeStruct((B, 1, H, D), jnp.float32),
        scratch_shapes=[
            pltpu.VMEM((B, H, CW), jnp.float32),
            pltpu.VMEM((B, H, CW), jnp.float32),
            pltpu.SemaphoreType.DMA((B,)),
            pltpu.SemaphoreType.DMA((B,)),
        ],
        compiler_params=pltpu.CompilerParams(collective_id=0),
    )(Q, Kt, Vt)
